# baseline (speedup 1.0000x reference)
import jax
import jax.numpy as jnp
from jax import lax
from jax.experimental import pallas as pl
from jax.experimental.pallas import tpu as pltpu
from jax.experimental.pallas import tpu_sc as plsc

_N_EMBD = 768
_N_TOKEN = 77
_BATCH = 1024
_LANES = 16
_PANEL = 128
_NPANEL = _N_EMBD // _PANEL
_BLK = 32
_NBLK = _BATCH // _BLK
_NUM_CORES = 2
_NUM_SUBCORES = 16
_NW = _NUM_CORES * _NUM_SUBCORES
_PAIRS_PER_W = _N_TOKEN * _NBLK // _NW


def _embed_body(tok_hbm, table_hbm, pos_hbm, out_hbm,
                pos_v, idx_v, b0, b1, b2, b3, b4, b5,
                g0, g1, g2, g3, g4, g5, w0, w1, w2, w3, w4, w5, si):
    bufs = (b0, b1, b2, b3, b4, b5)
    gsem = (g0, g1, g2, g3, g4, g5)
    wsem = (w0, w1, w2, w3, w4, w5)

    wid = lax.axis_index("subcore") * _NUM_CORES + lax.axis_index("core")
    base_pair = wid * _PAIRS_PER_W

    pltpu.sync_copy(pos_hbm, pos_v)
    pltpu.async_copy(
        tok_hbm.at[pl.ds(base_pair * _BLK, _PAIRS_PER_W * _BLK)],
        idx_v, si).wait()

    def g_start(q, cb):
        pltpu.async_copy(
            table_hbm.at[idx_v.at[pl.ds(q * _BLK, _BLK)],
                         pl.ds(cb * _PANEL, _PANEL)],
            bufs[cb], gsem[cb])

    def g_wait(q, cb):
        pltpu.make_async_copy(
            table_hbm.at[idx_v.at[pl.ds(q * _BLK, _BLK)],
                         pl.ds(cb * _PANEL, _PANEL)],
            bufs[cb], gsem[cb]).wait()

    def w_pair(q, cb):
        pair = base_pair + q
        t = pair // _NBLK
        bb = lax.rem(pair, _NBLK)
        return (bufs[cb],
                out_hbm.at[t, pl.ds(bb * _BLK, _BLK),
                           pl.ds(cb * _PANEL, _PANEL)])

    def w_start(q, cb):
        src, dst = w_pair(q, cb)
        pltpu.async_copy(src, dst, wsem[cb])

    def w_wait(q, cb):
        src, dst = w_pair(q, cb)
        pltpu.make_async_copy(src, dst, wsem[cb]).wait()

    for cb in (0, 1, 2, 3, 4):
        g_start(0, cb)

    @pl.loop(0, _PAIRS_PER_W)
    def _(q):
        t = (base_pair + q) // _NBLK

        for cb in range(_NPANEL):
            g_wait(q, cb)

            if cb < 1:
                jp = cb + 5

                @pl.when(q >= 1)
                def _():
                    w_wait(q - 1, jp)

                g_start(q, jp)
            else:
                jp = cb - 1
                w_wait(q, jp)

                @pl.when(q <= _PAIRS_PER_W - 2)
                def _():
                    g_start(q + 1, jp)

            vals = [pos_v[t, pl.ds(cb * _PANEL + c, _LANES)]
                    for c in range(0, _PANEL, _LANES)]

            @pl.loop(0, _BLK)
            def _(r):
                for k, c in enumerate(range(0, _PANEL, _LANES)):
                    plsc.addupdate(bufs[cb].at[r, pl.ds(c, _LANES)], vals[k])

            w_start(q, cb)

    w_wait(_PAIRS_PER_W - 1, 5)


@jax.jit
def _embed(tokens, token_embedding, position_embedding):
    tok_flat = tokens.T.reshape(_N_TOKEN * _BATCH)
    mesh = plsc.VectorSubcoreMesh(
        core_axis_name="core", subcore_axis_name="subcore")
    kern = pl.kernel(
        _embed_body,
        out_type=jax.ShapeDtypeStruct((_N_TOKEN, _BATCH, _N_EMBD),
                                      jnp.float32),
        mesh=mesh,
        scratch_types=(
            [pltpu.VMEM((_N_TOKEN, _N_EMBD), jnp.float32),
             pltpu.VMEM((_PAIRS_PER_W * _BLK,), jnp.int32)]
            + [pltpu.VMEM((_BLK, _PANEL), jnp.float32)
               for _ in range(_NPANEL)]
            + [pltpu.SemaphoreType.DMA for _ in range(2 * _NPANEL + 1)]
        ),
    )
    out_t = kern(tok_flat, token_embedding, position_embedding)
    return jnp.transpose(out_t, (1, 0, 2))


def kernel(tokens, token_embedding, position_embedding):
    return _embed(tokens, token_embedding, position_embedding)


# speedup vs baseline: 1.1275x; 1.1275x over previous
import jax
import jax.numpy as jnp
from jax import lax
from jax.experimental import pallas as pl
from jax.experimental.pallas import tpu as pltpu
from jax.experimental.pallas import tpu_sc as plsc

_N_EMBD = 768
_N_TOKEN = 77
_BATCH = 1024
_LANES = 16
_PANEL = 128
_NPANEL = _N_EMBD // _PANEL
_BLK = 32
_NBLK = _BATCH // _BLK
_NUM_CORES = 2
_NUM_SUBCORES = 16
_NW = _NUM_CORES * _NUM_SUBCORES
_PAIRS_PER_W = _N_TOKEN * _NBLK // _NW


def _embed_body(tok_hbm, table_hbm, pos_hbm, out_hbm, pos_v, idx_v,
                *rest):
    bufs = rest[:12]
    gsem = rest[12:24]
    wsem = rest[24:36]
    si = rest[36]

    wid = lax.axis_index("subcore") * _NUM_CORES + lax.axis_index("core")
    base_pair = wid * _PAIRS_PER_W

    pltpu.sync_copy(pos_hbm, pos_v)
    pltpu.async_copy(
        tok_hbm.at[pl.ds(base_pair * _BLK, _PAIRS_PER_W * _BLK)],
        idx_v, si).wait()

    def slot(q_parity, cb):
        return q_parity * _NPANEL + cb

    def g_start(q, s):
        pltpu.async_copy(
            table_hbm.at[idx_v.at[pl.ds(q * _BLK, _BLK)],
                         pl.ds((s % _NPANEL) * _PANEL, _PANEL)],
            bufs[s], gsem[s])

    def g_wait(q, s):
        pltpu.make_async_copy(
            table_hbm.at[idx_v.at[pl.ds(q * _BLK, _BLK)],
                         pl.ds((s % _NPANEL) * _PANEL, _PANEL)],
            bufs[s], gsem[s]).wait()

    def w_pair(q, s):
        pair = base_pair + q
        t = pair // _NBLK
        bb = lax.rem(pair, _NBLK)
        return (bufs[s],
                out_hbm.at[t, pl.ds(bb * _BLK, _BLK),
                           pl.ds((s % _NPANEL) * _PANEL, _PANEL)])

    def w_start(q, s):
        src, dst = w_pair(q, s)
        pltpu.async_copy(src, dst, wsem[s])

    def w_wait(q, s):
        src, dst = w_pair(q, s)
        pltpu.make_async_copy(src, dst, wsem[s]).wait()

    def chunk(q, h, cb, q2):
        s = slot(h, cb)
        sn = slot(1 - h, cb)
        g_wait(q, s)

        if h == 0:
            @pl.when(q2 >= 1)
            def _():
                w_wait(q - 1, sn)
        else:
            w_wait(q - 1, sn)

        @pl.when(q2 <= _PAIRS_PER_W // 2 - 1)
        def _():
            g_start(q + 1, sn)

        t = (base_pair + q) // _NBLK
        vals = [pos_v[t, pl.ds(cb * _PANEL + c, _LANES)]
                for c in range(0, _PANEL, _LANES)]

        @pl.loop(0, _BLK)
        def _(r):
            for k, c in enumerate(range(0, _PANEL, _LANES)):
                plsc.addupdate(bufs[s].at[r, pl.ds(c, _LANES)], vals[k])

        w_start(q, s)

    for cb in range(_NPANEL):
        g_start(0, slot(0, cb))

    @pl.loop(0, (_PAIRS_PER_W + 1) // 2)
    def _(q2):
        for cb in range(_NPANEL):
            chunk(2 * q2, 0, cb, q2)
        @pl.when(q2 <= _PAIRS_PER_W // 2 - 1)
        def _():
            for cb in range(_NPANEL):
                chunk(2 * q2 + 1, 1, cb, q2)

    for cb in range(_NPANEL):
        w_wait(_PAIRS_PER_W - 1, slot(0, cb))


@jax.jit
def _embed(tokens, token_embedding, position_embedding):
    tok_flat = tokens.T.reshape(_N_TOKEN * _BATCH)
    mesh = plsc.VectorSubcoreMesh(
        core_axis_name="core", subcore_axis_name="subcore")
    kern = pl.kernel(
        _embed_body,
        out_type=jax.ShapeDtypeStruct((_N_TOKEN, _BATCH, _N_EMBD),
                                      jnp.float32),
        mesh=mesh,
        scratch_types=(
            [pltpu.VMEM((_N_TOKEN, _N_EMBD), jnp.float32),
             pltpu.VMEM((_PAIRS_PER_W * _BLK,), jnp.int32)]
            + [pltpu.VMEM((_BLK, _PANEL), jnp.float32) for _ in range(12)]
            + [pltpu.SemaphoreType.DMA for _ in range(25)]
        ),
    )
    out_t = kern(tok_flat, token_embedding, position_embedding)
    return jnp.transpose(out_t, (1, 0, 2))


def kernel(tokens, token_embedding, position_embedding):
    return _embed(tokens, token_embedding, position_embedding)
